# baseline (speedup 1.0000x reference)
import functools

import jax
import jax.numpy as jnp
from jax import lax
from jax.experimental import pallas as pl
from jax.experimental.pallas import tpu as pltpu
from jax.experimental.pallas import tpu_sc as plsc

_GROUP = 4
_NGROUPS = 20
_MDIMS = 80
_BATCH = 16384
_NC, _NS, _L = 2, 16, 16
_NW = _NC * _NS
_ROWS_W = _BATCH // _NW
_XW = _ROWS_W * _NGROUPS
_OW = _ROWS_W * _MDIMS
_UNROLL = 8


def _sc_body(x_hbm, out_hbm, x_v, y_v):
    wid = lax.axis_index("s") * _NC + lax.axis_index("c")
    pltpu.sync_copy(x_hbm.at[pl.ds(wid * _XW, _XW)], x_v)
    lane = lax.iota(jnp.int32, _L)
    rep = lax.shift_right_logical(lane, jnp.full((_L,), 2, jnp.int32))
    lo = jnp.full((_L,), -1.0, jnp.float32)
    hi = jnp.full((_L,), 1.0, jnp.float32)

    def step(m, carry):
        for u in range(_UNROLL):
            k = m * _UNROLL + u
            idx = jnp.full((_L,), k * _GROUP, jnp.int32) + rep
            v = plsc.load_gather(x_v, [idx])
            v = jnp.minimum(jnp.maximum(v, lo), hi)
            y_v[pl.ds(k * _L, _L)] = v
        return carry

    lax.fori_loop(0, _OW // (_L * _UNROLL), step, 0)
    pltpu.sync_copy(y_v, out_hbm.at[pl.ds(wid * _OW, _OW)])


@functools.partial(jax.jit, static_argnames=())
def _sc_expand(x_flat):
    mesh = plsc.VectorSubcoreMesh(core_axis_name="c", subcore_axis_name="s",
                                  num_cores=_NC, num_subcores=_NS)
    return pl.kernel(
        _sc_body,
        out_type=jax.ShapeDtypeStruct((_BATCH * _MDIMS,), jnp.float32),
        mesh=mesh,
        scratch_types=[
            pltpu.VMEM((_XW,), jnp.float32),
            pltpu.VMEM((_OW,), jnp.float32),
        ],
        compiler_params=pltpu.CompilerParams(needs_layout_passes=False),
    )(x_flat)


def kernel(x, latent_pi, W, b, noise):
    del latent_pi, W, b, noise
    out_flat = _sc_expand(x.reshape(-1))
    return out_flat.reshape(x.shape[0], _MDIMS)


# speedup vs baseline: 1.2143x; 1.2143x over previous
import functools

import jax
import jax.numpy as jnp
from jax import lax
from jax.experimental import pallas as pl
from jax.experimental.pallas import tpu as pltpu
from jax.experimental.pallas import tpu_sc as plsc

_GROUP = 4
_NGROUPS = 20
_MDIMS = 80
_BATCH = 16384
_NC, _NS, _L = 2, 16, 16
_NW = _NC * _NS
_ROWS_W = _BATCH // _NW
_XW = _ROWS_W * _NGROUPS
_OW = _ROWS_W * _MDIMS
_UNROLL = 8


def _sc_body(x_hbm, out_hbm, x_v, y_v):
    wid = lax.axis_index("s") * _NC + lax.axis_index("c")
    pltpu.sync_copy(x_hbm.at[pl.ds(wid * _XW, _XW)], x_v)
    lane = lax.iota(jnp.int32, _L)
    rep = lax.shift_right_logical(lane, jnp.full((_L,), 2, jnp.int32))
    lo = jnp.full((_L,), -1.0, jnp.float32)
    hi = jnp.full((_L,), 1.0, jnp.float32)

    @plsc.parallel_loop(0, _OW // _L, 1, unroll=_UNROLL)
    def _expand(k):
        idx = jnp.full((_L,), k * _GROUP, jnp.int32) + rep
        v = plsc.load_gather(x_v, [idx])
        v = jnp.minimum(jnp.maximum(v, lo), hi)
        y_v[pl.ds(k * _L, _L)] = v
    pltpu.sync_copy(y_v, out_hbm.at[pl.ds(wid * _OW, _OW)])


@functools.partial(jax.jit, static_argnames=())
def _sc_expand(x_flat):
    mesh = plsc.VectorSubcoreMesh(core_axis_name="c", subcore_axis_name="s",
                                  num_cores=_NC, num_subcores=_NS)
    return pl.kernel(
        _sc_body,
        out_type=jax.ShapeDtypeStruct((_BATCH * _MDIMS,), jnp.float32),
        mesh=mesh,
        scratch_types=[
            pltpu.VMEM((_XW,), jnp.float32),
            pltpu.VMEM((_OW,), jnp.float32),
        ],
        compiler_params=pltpu.CompilerParams(needs_layout_passes=False),
    )(x_flat)


def kernel(x, latent_pi, W, b, noise):
    del latent_pi, W, b, noise
    out_flat = _sc_expand(x.reshape(-1))
    return out_flat.reshape(x.shape[0], _MDIMS)


# speedup vs baseline: 1.7391x; 1.4322x over previous
import functools

import jax
import jax.numpy as jnp
from jax import lax
from jax.experimental import pallas as pl
from jax.experimental.pallas import tpu as pltpu
from jax.experimental.pallas import tpu_sc as plsc

_GROUP = 4
_NGROUPS = 20
_MDIMS = 80
_BATCH = 16384
_NC, _NS, _L = 2, 16, 16
_NW = _NC * _NS
_ROWS_W = _BATCH // _NW
_VPR = _MDIMS // _L
_UNROLL = 2


def _sc_body(x_hbm, out_hbm, x_v, y_v):
    wid = lax.axis_index("s") * _NC + lax.axis_index("c")
    r0 = wid * _ROWS_W
    pltpu.sync_copy(x_hbm.at[pl.ds(r0, _ROWS_W)], x_v)
    lane = lax.iota(jnp.int32, _L)
    rep = lax.shift_right_logical(lane, jnp.full((_L,), 2, jnp.int32))
    cols = [rep + jnp.full((_L,), j * _GROUP, jnp.int32) for j in range(_VPR)]
    lo = jnp.full((_L,), -1.0, jnp.float32)
    hi = jnp.full((_L,), 1.0, jnp.float32)

    @plsc.parallel_loop(0, _ROWS_W, 1, unroll=_UNROLL)
    def _expand(r):
        row = jnp.full((_L,), r, jnp.int32)
        for j in range(_VPR):
            v = plsc.load_gather(x_v, [row, cols[j]])
            v = jnp.minimum(jnp.maximum(v, lo), hi)
            y_v[r, pl.ds(j * _L, _L)] = v

    pltpu.sync_copy(y_v, out_hbm.at[pl.ds(r0, _ROWS_W)])


@functools.partial(jax.jit, static_argnames=())
def _sc_expand(x):
    mesh = plsc.VectorSubcoreMesh(core_axis_name="c", subcore_axis_name="s",
                                  num_cores=_NC, num_subcores=_NS)
    return pl.kernel(
        _sc_body,
        out_type=jax.ShapeDtypeStruct((_BATCH, _MDIMS), jnp.float32),
        mesh=mesh,
        scratch_types=[
            pltpu.VMEM((_ROWS_W, _NGROUPS), jnp.float32),
            pltpu.VMEM((_ROWS_W, _MDIMS), jnp.float32),
        ],
        compiler_params=pltpu.CompilerParams(needs_layout_passes=False),
    )(x)


def kernel(x, latent_pi, W, b, noise):
    del latent_pi, W, b, noise
    return _sc_expand(x)


# speedup vs baseline: 1.7532x; 1.0081x over previous
import functools

import jax
import jax.numpy as jnp
from jax import lax
from jax.experimental import pallas as pl
from jax.experimental.pallas import tpu as pltpu
from jax.experimental.pallas import tpu_sc as plsc

_GROUP = 4
_NGROUPS = 20
_MDIMS = 80
_BATCH = 16384
_NC, _NS, _L = 2, 16, 16
_NW = _NC * _NS
_ROWS_W = _BATCH // _NW
_VPR = _MDIMS // _L
_NCHUNK = 4
_CROWS = _ROWS_W // _NCHUNK
_UNROLL = 8


def _sc_body(x_hbm, out_hbm, x_v, y0, y1, sem0, sem1):
    wid = lax.axis_index("s") * _NC + lax.axis_index("c")
    r0 = wid * _ROWS_W
    pltpu.sync_copy(x_hbm.at[pl.ds(r0, _ROWS_W)], x_v)
    lane = lax.iota(jnp.int32, _L)
    rep = lax.shift_right_logical(lane, jnp.full((_L,), 2, jnp.int32))
    cols = [rep + jnp.full((_L,), j * _GROUP, jnp.int32) for j in range(_VPR)]
    lo = jnp.full((_L,), -1.0, jnp.float32)
    hi = jnp.full((_L,), 1.0, jnp.float32)
    bufs = (y0, y1)
    sems = (sem0, sem1)
    copies = [None] * _NCHUNK

    for c in range(_NCHUNK):
        buf = bufs[c % 2]
        if c >= 2:
            copies[c - 2].wait()

        @plsc.parallel_loop(0, _CROWS, 1, unroll=_UNROLL)
        def _expand(r):
            row = jnp.full((_L,), c * _CROWS + r, jnp.int32)
            for j in range(_VPR):
                v = plsc.load_gather(x_v, [row, cols[j]])
                v = jnp.minimum(jnp.maximum(v, lo), hi)
                buf[r, pl.ds(j * _L, _L)] = v

        copies[c] = pltpu.async_copy(
            buf, out_hbm.at[pl.ds(r0 + c * _CROWS, _CROWS)], sems[c % 2])

    copies[_NCHUNK - 2].wait()
    copies[_NCHUNK - 1].wait()


@functools.partial(jax.jit, static_argnames=())
def _sc_expand(x):
    mesh = plsc.VectorSubcoreMesh(core_axis_name="c", subcore_axis_name="s",
                                  num_cores=_NC, num_subcores=_NS)
    return pl.kernel(
        _sc_body,
        out_type=jax.ShapeDtypeStruct((_BATCH, _MDIMS), jnp.float32),
        mesh=mesh,
        scratch_types=[
            pltpu.VMEM((_ROWS_W, _NGROUPS), jnp.float32),
            pltpu.VMEM((_CROWS, _MDIMS), jnp.float32),
            pltpu.VMEM((_CROWS, _MDIMS), jnp.float32),
            pltpu.SemaphoreType.DMA,
            pltpu.SemaphoreType.DMA,
        ],
        compiler_params=pltpu.CompilerParams(needs_layout_passes=False),
    )(x)


def kernel(x, latent_pi, W, b, noise):
    del latent_pi, W, b, noise
    return _sc_expand(x)


# speedup vs baseline: 1.7617x; 1.0048x over previous
import functools

import jax
import jax.numpy as jnp
from jax import lax
from jax.experimental import pallas as pl
from jax.experimental.pallas import tpu as pltpu
from jax.experimental.pallas import tpu_sc as plsc

_GROUP = 4
_NGROUPS = 20
_MDIMS = 80
_BATCH = 16384
_NC, _NS, _L = 2, 16, 16
_NW = _NC * _NS
_ROWS_W = _BATCH // _NW
_VPR = _MDIMS // _L
_NCHUNK = 4
_CROWS = _ROWS_W // _NCHUNK
_UNROLL = 8

_GDN = lax.GatherDimensionNumbers(
    offset_dims=(), collapsed_slice_dims=(0,), start_index_map=(0,))


def _permute(v, idx):
    return lax.gather(v, idx[:, None], _GDN, slice_sizes=(1,),
                      mode=lax.GatherScatterMode.PROMISE_IN_BOUNDS)


def _sc_body(x_hbm, out_hbm, x_v, y0, y1, sem0, sem1):
    wid = lax.axis_index("s") * _NC + lax.axis_index("c")
    r0 = wid * _ROWS_W
    pltpu.sync_copy(x_hbm.at[pl.ds(r0, _ROWS_W)], x_v)
    lane = lax.iota(jnp.int32, _L)
    rep = lax.shift_right_logical(lane, jnp.full((_L,), 2, jnp.int32))
    perms = [rep + jnp.full((_L,), 4 * j, jnp.int32) for j in range(4)]
    perm_b = rep + jnp.full((_L,), 12, jnp.int32)
    lo = jnp.full((_L,), -1.0, jnp.float32)
    hi = jnp.full((_L,), 1.0, jnp.float32)
    bufs = (y0, y1)
    sems = (sem0, sem1)
    copies = [None] * _NCHUNK

    for c in range(_NCHUNK):
        buf = bufs[c % 2]
        if c >= 2:
            copies[c - 2].wait()

        @plsc.parallel_loop(0, _CROWS, 1, unroll=_UNROLL)
        def _expand(r):
            row = c * _CROWS + r
            va = x_v[row, pl.ds(0, _L)]
            vb = x_v[row, pl.ds(_NGROUPS - _L, _L)]
            for j in range(_VPR):
                v = _permute(vb, perm_b) if j == 4 else _permute(va, perms[j])
                v = jnp.minimum(jnp.maximum(v, lo), hi)
                buf[r, pl.ds(j * _L, _L)] = v

        copies[c] = pltpu.async_copy(
            buf, out_hbm.at[pl.ds(r0 + c * _CROWS, _CROWS)], sems[c % 2])

    copies[_NCHUNK - 2].wait()
    copies[_NCHUNK - 1].wait()


@functools.partial(jax.jit, static_argnames=())
def _sc_expand(x):
    mesh = plsc.VectorSubcoreMesh(core_axis_name="c", subcore_axis_name="s",
                                  num_cores=_NC, num_subcores=_NS)
    return pl.kernel(
        _sc_body,
        out_type=jax.ShapeDtypeStruct((_BATCH, _MDIMS), jnp.float32),
        mesh=mesh,
        scratch_types=[
            pltpu.VMEM((_ROWS_W, _NGROUPS), jnp.float32),
            pltpu.VMEM((_CROWS, _MDIMS), jnp.float32),
            pltpu.VMEM((_CROWS, _MDIMS), jnp.float32),
            pltpu.SemaphoreType.DMA,
            pltpu.SemaphoreType.DMA,
        ],
        compiler_params=pltpu.CompilerParams(needs_layout_passes=False,
                                             use_tc_tiling_on_sc=True),
    )(x)


def kernel(x, latent_pi, W, b, noise):
    del latent_pi, W, b, noise
    return _sc_expand(x)


# speedup vs baseline: 2.5891x; 1.4697x over previous
import functools

import jax
import jax.numpy as jnp
from jax import lax
from jax.experimental import pallas as pl
from jax.experimental.pallas import tpu as pltpu
from jax.experimental.pallas import tpu_sc as plsc

_GROUP = 4
_NGROUPS = 20
_MDIMS = 80
_BATCH = 16384
_NC, _NS, _L = 2, 16, 16
_NW = _NC * _NS
_COLS_W = _BATCH // _NW
_VPC = _COLS_W // _L
_UNROLL = 4


def _sc_body(xt_hbm, outt_hbm, x_v, y_v):
    wid = lax.axis_index("s") * _NC + lax.axis_index("c")
    c0 = wid * _COLS_W
    pltpu.sync_copy(xt_hbm.at[:, pl.ds(c0, _COLS_W)], x_v)
    lo = jnp.full((_L,), -1.0, jnp.float32)
    hi = jnp.full((_L,), 1.0, jnp.float32)

    @plsc.parallel_loop(0, _VPC, 1, unroll=_UNROLL)
    def _col(v):
        col = v * _L
        for i in range(_NGROUPS):
            xv = x_v[i, pl.ds(col, _L)]
            xv = jnp.minimum(jnp.maximum(xv, lo), hi)
            for j in range(_GROUP):
                y_v[_GROUP * i + j, pl.ds(col, _L)] = xv

    pltpu.sync_copy(y_v, outt_hbm.at[:, pl.ds(c0, _COLS_W)])


@functools.partial(jax.jit, static_argnames=())
def _sc_expand(x_t):
    mesh = plsc.VectorSubcoreMesh(core_axis_name="c", subcore_axis_name="s",
                                  num_cores=_NC, num_subcores=_NS)
    return pl.kernel(
        _sc_body,
        out_type=jax.ShapeDtypeStruct((_MDIMS, _BATCH), jnp.float32),
        mesh=mesh,
        scratch_types=[
            pltpu.VMEM((_NGROUPS, _COLS_W), jnp.float32),
            pltpu.VMEM((_MDIMS, _COLS_W), jnp.float32),
        ],
        compiler_params=pltpu.CompilerParams(needs_layout_passes=False),
    )(x_t)


def kernel(x, latent_pi, W, b, noise):
    del latent_pi, W, b, noise
    return _sc_expand(x.T).T
